# baseline (speedup 1.0000x reference)
import functools

import jax
import jax.numpy as jnp
from jax import lax
from jax.experimental import pallas as pl
from jax.experimental.pallas import tpu as pltpu
from jax.experimental.pallas import tpu_sc as plsc

ORDER = 16
SUBSAMPLING_FACTOR = 2
NUM_FEATURES = 64
SUB_ORDER = ORDER // SUBSAMPLING_FACTOR

NC = 2
NS = 16
NW = NC * NS

CCH = 4
NBUF = 3
DEPTH = 2


def _make_sc_copy(B, H, W):
    mesh = plsc.VectorSubcoreMesh(core_axis_name="c", subcore_axis_name="s")
    n_blocks = B * SUB_ORDER
    blocks_per_w = n_blocks // NW
    chunks_per_block = NUM_FEATURES // CCH
    n_tr = blocks_per_w * chunks_per_block

    @functools.partial(
        pl.kernel,
        mesh=mesh,
        out_type=jax.ShapeDtypeStruct((B, SUB_ORDER * NUM_FEATURES, H, W), jnp.float32),
        scratch_types=[pltpu.VMEM((1, CCH, H, W), jnp.float32) for _ in range(NBUF)]
        + [pltpu.SemaphoreType.DMA] * (2 * NBUF),
    )
    def k(x_hbm, out_hbm, *bufs_and_sems):
        bufs = bufs_and_sems[:NBUF]
        sin = bufs_and_sems[NBUF : 2 * NBUF]
        sout = bufs_and_sems[2 * NBUF :]
        wid = lax.axis_index("s") * NC + lax.axis_index("c")

        def mk(t):
            blk = wid * blocks_per_w + t // chunks_per_block
            b = blk // SUB_ORDER
            g = blk % SUB_ORDER
            ch = (t % chunks_per_block) * CCH
            r = t % NBUF
            cin = pltpu.make_async_copy(
                x_hbm.at[
                    pl.ds(b, 1),
                    pl.ds(g * SUBSAMPLING_FACTOR * NUM_FEATURES + ch, CCH),
                ],
                bufs[r],
                sin[r],
            )
            cout = pltpu.make_async_copy(
                bufs[r],
                out_hbm.at[pl.ds(b, 1), pl.ds(g * NUM_FEATURES + ch, CCH)],
                sout[r],
            )
            return cin, cout

        copies = [mk(t) for t in range(n_tr)]
        out_waited = [False] * n_tr
        for t in range(min(DEPTH, n_tr)):
            copies[t][0].start()
        for t in range(n_tr):
            copies[t][0].wait()
            copies[t][1].start()
            u = t + DEPTH
            if u < n_tr:
                if u >= NBUF:
                    copies[u - NBUF][1].wait()
                    out_waited[u - NBUF] = True
                copies[u][0].start()
        for t in range(n_tr):
            if not out_waited[t]:
                copies[t][1].wait()

    return k


def kernel(x):
    B, C, H, W = x.shape
    return _make_sc_copy(B, H, W)(x)
